# baseline (speedup 1.0000x reference)
import functools

import jax
import jax.numpy as jnp
import numpy as np
from jax import lax
from jax.experimental import pallas as pl
from jax.experimental.pallas import tpu as pltpu
from jax.experimental.pallas import tpu_sc as plsc

_BS, _CH = 16, 768
_HW = 1024
_NP = _BS // 2
_TPP = 4
_RPT = _HW // _TPP
_R = 16
_NBLK = _RPT // _R
_NV = _CH // 16


def _build_masks():
    rng = np.random.default_rng(0)
    r_lo, r_hi = 0.1, 0.3
    rng.random()
    sel = np.zeros((_NP, _CH), np.float32)
    for p, i in enumerate(range(0, _BS - 1, 2)):
        num_first = int(_CH * (rng.random() * (r_hi - r_lo) + r_lo))
        perm = rng.permutation(_CH)
        sel[p, perm[:num_first]] = 1.0
    rng.random()
    nf = int(_CH * (rng.random() * (r_hi - r_lo) + r_lo))
    ns = int(_CH * (rng.random() * (r_hi - r_lo) + r_lo))
    perm = rng.permutation(_CH)
    keep = np.ones(_CH, np.float32)
    keep[perm[nf:nf + ns]] = 0.0
    a = keep[None, :] * (1.0 - sel)
    b = keep[None, :] * sel
    k = np.tile(keep[None, :], (_NP, 1))
    return np.stack([a, b, k], axis=1).astype(np.float32)


_MASKS = _build_masks()


def _compute_block(ye, yo, masks_v):
    def vbody(v, carry):
        sl = pl.ds(v * 16, 16)
        va = masks_v[0, sl]
        vb = masks_v[1, sl]
        vk = masks_v[2, sl]

        def rbody(r):
            e = ye[r, sl]
            o = yo[r, sl]
            ye[r, sl] = e * va + o * vb
            yo[r, sl] = o * vk

        plsc.parallel_loop(0, _R, 1, unroll=16)(rbody)
        return carry

    lax.fori_loop(0, _NV, vbody, 0)


_NSETS = 4
_PRIME = _NSETS - 1


def _sc_body(y_hbm, masks_hbm, out_hbm, bufs, masks_v, insems, outsems):
    wid = lax.axis_index("s") * 2 + lax.axis_index("c")
    p = wid // _TPP
    st = lax.rem(wid, _TPP)
    base_e = (2 * p) * _HW + st * _RPT
    base_o = base_e + _HW

    pltpu.sync_copy(masks_hbm.at[p], masks_v)

    in_h = [None] * _NSETS
    out_h = [None] * _NSETS

    def start_in(k):
        s = k % _NSETS
        ye, yo = bufs[s]
        in_h[s] = (
            pltpu.async_copy(y_hbm.at[pl.ds(base_e + k * _R, _R)], ye, insems[s]),
            pltpu.async_copy(y_hbm.at[pl.ds(base_o + k * _R, _R)], yo, insems[s]),
        )

    for k in range(min(_PRIME, _NBLK)):
        start_in(k)
    for k in range(_NBLK):
        s = k % _NSETS
        ye, yo = bufs[s]
        for h in in_h[s]:
            h.wait()
        _compute_block(ye, yo, masks_v)
        out_h[s] = (
            pltpu.async_copy(ye, out_hbm.at[pl.ds(base_e + k * _R, _R)], outsems[s]),
            pltpu.async_copy(yo, out_hbm.at[pl.ds(base_o + k * _R, _R)], outsems[s]),
        )
        if k + _PRIME < _NBLK:
            nxt = (k + _PRIME) % _NSETS
            if out_h[nxt] is not None:
                for h in out_h[nxt]:
                    h.wait()
                out_h[nxt] = None
            start_in(k + _PRIME)
    for hs in out_h:
        if hs is not None:
            for h in hs:
                h.wait()


@functools.partial(
    pl.kernel,
    out_type=jax.ShapeDtypeStruct((_BS * _HW, _CH), jnp.float32),
    mesh=plsc.VectorSubcoreMesh(core_axis_name="c", subcore_axis_name="s"),
    scratch_types=(
        [pltpu.VMEM((_R, _CH), jnp.float32) for _ in range(2 * _NSETS)]
        + [pltpu.VMEM((3, _CH), jnp.float32)]
        + [pltpu.SemaphoreType.DMA for _ in range(2 * _NSETS)]
    ),
)
def _content_fa_sc(y_hbm, masks_hbm, out_hbm, *scratch):
    data = scratch[: 2 * _NSETS]
    masks_v = scratch[2 * _NSETS]
    sems = scratch[2 * _NSETS + 1:]
    bufs = tuple((data[2 * s], data[2 * s + 1]) for s in range(_NSETS))
    insems = sems[:_NSETS]
    outsems = sems[_NSETS:]
    _sc_body(y_hbm, masks_hbm, out_hbm, bufs, masks_v, insems, outsems)


def kernel(y, epoch):
    del epoch
    y_t = jnp.transpose(y, (0, 2, 3, 1))
    y2 = jnp.reshape(y_t, (_BS * _HW, _CH))
    out = _content_fa_sc(y2, jnp.asarray(_MASKS))
    out_t = jnp.reshape(out, (_BS, 32, 32, _CH))
    return jnp.transpose(out_t, (0, 3, 1, 2))


# speedup vs baseline: 1.0343x; 1.0343x over previous
import functools

import jax
import jax.numpy as jnp
import numpy as np
from jax import lax
from jax.experimental import pallas as pl
from jax.experimental.pallas import tpu as pltpu
from jax.experimental.pallas import tpu_sc as plsc

_BS, _CH = 16, 768
_HW = 1024
_NP = _BS // 2
_TPP = 4
_RPT = _HW // _TPP
_R = 16
_NBLK = _RPT // _R
_NV = _CH // 16


def _build_masks():
    rng = np.random.default_rng(0)
    r_lo, r_hi = 0.1, 0.3
    rng.random()
    sel = np.zeros((_NP, _CH), np.float32)
    for p, i in enumerate(range(0, _BS - 1, 2)):
        num_first = int(_CH * (rng.random() * (r_hi - r_lo) + r_lo))
        perm = rng.permutation(_CH)
        sel[p, perm[:num_first]] = 1.0
    rng.random()
    nf = int(_CH * (rng.random() * (r_hi - r_lo) + r_lo))
    ns = int(_CH * (rng.random() * (r_hi - r_lo) + r_lo))
    perm = rng.permutation(_CH)
    keep = np.ones(_CH, np.float32)
    keep[perm[nf:nf + ns]] = 0.0
    a = keep[None, :] * (1.0 - sel)
    b = keep[None, :] * sel
    k = np.tile(keep[None, :], (_NP, 1))
    return np.stack([a, b, k], axis=1).astype(np.float32)


_MASKS = _build_masks()


def _compute_block(ye, yo, masks_v):
    def vbody(v, carry):
        sl = pl.ds(v * 16, 16)
        va = masks_v[0, sl]
        vb = masks_v[1, sl]
        vk = masks_v[2, sl]

        def rbody(r):
            e = ye[r, sl]
            o = yo[r, sl]
            ye[r, sl] = e * va + o * vb
            yo[r, sl] = o * vk

        plsc.parallel_loop(0, _R, 1, unroll=8)(rbody)
        return carry

    lax.fori_loop(0, _NV, vbody, 0)


_NSETS = 4
_PRIME = _NSETS - 1


def _sc_body(y_hbm, masks_hbm, out_hbm, bufs, masks_v, insems, outsems):
    wid = lax.axis_index("s") * 2 + lax.axis_index("c")
    p = wid // _TPP
    st = lax.rem(wid, _TPP)
    base_e = (2 * p) * _HW + st * _RPT
    base_o = base_e + _HW

    in_h = [None] * _NSETS
    out_h = [None] * _NSETS

    def start_in(k):
        s = k % _NSETS
        ye, yo = bufs[s]
        in_h[s] = (
            pltpu.async_copy(y_hbm.at[pl.ds(base_e + k * _R, _R)], ye, insems[s]),
            pltpu.async_copy(y_hbm.at[pl.ds(base_o + k * _R, _R)], yo, insems[s]),
        )

    mask_h = pltpu.async_copy(masks_hbm.at[p], masks_v, insems[_NSETS - 1])
    for k in range(min(_PRIME, _NBLK)):
        start_in(k)
    mask_h.wait()
    for k in range(_NBLK):
        s = k % _NSETS
        ye, yo = bufs[s]
        for h in in_h[s]:
            h.wait()
        _compute_block(ye, yo, masks_v)
        out_h[s] = (
            pltpu.async_copy(ye, out_hbm.at[pl.ds(base_e + k * _R, _R)], outsems[s]),
            pltpu.async_copy(yo, out_hbm.at[pl.ds(base_o + k * _R, _R)], outsems[s]),
        )
        if k + _PRIME < _NBLK:
            nxt = (k + _PRIME) % _NSETS
            if out_h[nxt] is not None:
                for h in out_h[nxt]:
                    h.wait()
                out_h[nxt] = None
            start_in(k + _PRIME)
    for hs in out_h:
        if hs is not None:
            for h in hs:
                h.wait()


@functools.partial(
    pl.kernel,
    out_type=jax.ShapeDtypeStruct((_BS * _HW, _CH), jnp.float32),
    mesh=plsc.VectorSubcoreMesh(core_axis_name="c", subcore_axis_name="s"),
    scratch_types=(
        [pltpu.VMEM((_R, _CH), jnp.float32) for _ in range(2 * _NSETS)]
        + [pltpu.VMEM((3, _CH), jnp.float32)]
        + [pltpu.SemaphoreType.DMA for _ in range(2 * _NSETS)]
    ),
)
def _content_fa_sc(y_hbm, masks_hbm, out_hbm, *scratch):
    data = scratch[: 2 * _NSETS]
    masks_v = scratch[2 * _NSETS]
    sems = scratch[2 * _NSETS + 1:]
    bufs = tuple((data[2 * s], data[2 * s + 1]) for s in range(_NSETS))
    insems = sems[:_NSETS]
    outsems = sems[_NSETS:]
    _sc_body(y_hbm, masks_hbm, out_hbm, bufs, masks_v, insems, outsems)


def kernel(y, epoch):
    del epoch
    y_t = jnp.transpose(y, (0, 2, 3, 1))
    y2 = jnp.reshape(y_t, (_BS * _HW, _CH))
    out = _content_fa_sc(y2, jnp.asarray(_MASKS))
    out_t = jnp.reshape(out, (_BS, 32, 32, _CH))
    return jnp.transpose(out_t, (0, 3, 1, 2))
